# baseline (speedup 1.0000x reference)
import jax
import jax.numpy as jnp
from jax import lax
from jax.experimental import pallas as pl
from jax.experimental.pallas import tpu as pltpu
from jax.experimental.pallas import tpu_sc as plsc

N_NODES = 10000
N_EDGES = 160000
D = 256
DH = D // 2
NT = 16
EPT = N_EDGES // NT
B = 40
NCH = EPT // B
NBUF = 4
NWT = 10
RPT = N_NODES // NWT
ZR = 40


def _zero_fill(ref, nrows, ncols):
    def row(i, _):
        def col(j, _):
            ref[i, pl.ds(j * 16, 16)] = jnp.zeros((16,), jnp.float32)
            return 0
        lax.fori_loop(0, ncols // 16, col, 0)
        return 0
    lax.fori_loop(0, nrows, row, 0)


def _sc_body(x_lo, x_hi, ew_hbm, src_hbm, dst_hbm,
             summed_out,
             idx_s, idx_d, xg, ewv, zbuf,
             sidx, sdst, sew, sg, ssc,
             acc):
    c = lax.axis_index("c")
    s = lax.axis_index("s")
    row0 = pl.multiple_of(s * RPT, 8)

    @pl.when(s < NWT)
    def _():
        _zero_fill(zbuf, ZR, DH)
        for r in range(0, RPT, ZR):
            pltpu.sync_copy(zbuf, acc.at[pl.ds(row0 + r, ZR)])

    plsc.subcore_barrier()

    ebase = s * EPT

    def process(xref, col0):
        def fetch_src(j, k):
            base = pl.multiple_of(ebase + j * B, 8)
            pltpu.async_copy(src_hbm.at[pl.ds(base, B)], idx_s.at[k],
                             sidx.at[k])

        def fetch_dst(j, k):
            base = pl.multiple_of(ebase + j * B, 8)
            pltpu.async_copy(dst_hbm.at[pl.ds(base, B)], idx_d.at[k],
                             sdst.at[k])

        def fetch_ew(j, k):
            base = pl.multiple_of(ebase + j * B, 8)
            pltpu.async_copy(ew_hbm.at[pl.ds(base, B), pl.ds(col0, DH)],
                             ewv.at[k], sew.at[k])

        def gather_start(k):
            pltpu.make_async_copy(src_hbm.at[pl.ds(0, B)], idx_s.at[k],
                                  sidx.at[k]).wait()
            pltpu.async_copy(xref.at[idx_s.at[k]], xg.at[k], sg.at[k])

        def compute_scatter(k):
            pltpu.make_async_copy(ew_hbm.at[pl.ds(0, B), pl.ds(col0, DH)],
                                  ewv.at[k], sew.at[k]).wait()
            pltpu.make_async_copy(xref.at[pl.ds(0, B)], xg.at[k],
                                  sg.at[k]).wait()

            @plsc.parallel_loop(0, B, 1, unroll=4)
            def _(i):
                for jj in range(DH // 16):
                    sl = pl.ds(jj * 16, 16)
                    ewv[k, i, sl] = ewv[k, i, sl] * xg[k, i, sl]
            pltpu.make_async_copy(dst_hbm.at[pl.ds(0, B)], idx_d.at[k],
                                  sdst.at[k]).wait()
            pltpu.async_copy(ewv.at[k], acc.at[idx_d.at[k]], ssc.at[k],
                             add=True)

        def scatter_wait(k):
            pltpu.make_async_copy(ewv.at[k], acc.at[pl.ds(0, B)],
                                  ssc.at[k]).wait()

        for u in range(NBUF):
            fetch_src(u, u)
        for u in range(NBUF - 1):
            fetch_dst(u, u)
            fetch_ew(u, u)
        for u in range(NBUF - 1):
            gather_start(u)

        def quad(q, _):
            for u in range(NBUF):
                j = NBUF * q + u
                compute_scatter(u)

                @pl.when(j + 3 < NCH)
                def _():
                    gather_start((u + 3) % NBUF)

                @pl.when(j > 0)
                def _():
                    scatter_wait((u + 3) % NBUF)

                @pl.when(j + 3 < NCH)
                def _():
                    fetch_dst(j + 3, (u + 3) % NBUF)
                    fetch_ew(j + 3, (u + 3) % NBUF)

                @pl.when(j + 4 < NCH)
                def _():
                    fetch_src(j + 4, u)
            return 0
        lax.fori_loop(0, NCH // NBUF, quad, 0)
        compute_scatter((NCH - 2) % NBUF)
        scatter_wait((NCH - 3) % NBUF)
        compute_scatter((NCH - 1) % NBUF)
        scatter_wait((NCH - 2) % NBUF)
        scatter_wait((NCH - 1) % NBUF)

    @pl.when(c == 0)
    def _():
        process(x_lo, 0)

    @pl.when(c == 1)
    def _():
        process(x_hi, DH)

    plsc.subcore_barrier()

    @pl.when(s < NWT)
    def _():
        pltpu.sync_copy(acc.at[pl.ds(row0, RPT)],
                        summed_out.at[c, pl.ds(row0, RPT)])


_sc_call = pl.kernel(
    _sc_body,
    out_type=jax.ShapeDtypeStruct((2, N_NODES, DH), jnp.float32),
    mesh=plsc.VectorSubcoreMesh(core_axis_name="c", subcore_axis_name="s"),
    scratch_types=[
        pltpu.VMEM((NBUF, B), jnp.int32),
        pltpu.VMEM((NBUF, B), jnp.int32),
        pltpu.VMEM((NBUF, B, DH), jnp.float32),
        pltpu.VMEM((NBUF, B, DH), jnp.float32),
        pltpu.VMEM((ZR, DH), jnp.float32),
        pltpu.SemaphoreType.DMA((NBUF,)),
        pltpu.SemaphoreType.DMA((NBUF,)),
        pltpu.SemaphoreType.DMA((NBUF,)),
        pltpu.SemaphoreType.DMA((NBUF,)),
        pltpu.SemaphoreType.DMA((NBUF,)),
        pltpu.VMEM_SHARED((N_NODES, DH), jnp.float32),
    ],
    name="gcn_edge_sc",
)


EB = 16000
NEB = N_EDGES // EB
RB = 1000
NMM = N_NODES // RB


def _tc_body(dst_ref, slo_ref, shi_ref, x_ref, wl_ref, wr_ref, b_ref,
             out_ref, cnt):
    i = pl.program_id(0)

    @pl.when(i == 0)
    def _():
        cnt[...] = jnp.zeros_like(cnt)

    @pl.when(i < NEB)
    def _():
        val = dst_ref[...]
        j = lax.broadcasted_iota(jnp.int32, (1, 128), 1)
        oh_hi = (jnp.right_shift(val, 7) == j).astype(jnp.bfloat16)
        oh_lo = ((val & 127) == j).astype(jnp.bfloat16)
        cnt[...] += lax.dot_general(
            oh_hi, oh_lo, (((0,), (0,)), ((), ())),
            preferred_element_type=jnp.float32)

        @pl.when(i == NEB - 1)
        def _():
            cnt[...] = 1.0 / jnp.clip(cnt[...], 1.0, None)

    @pl.when(i >= NEB)
    def _():
        r = i - NEB
        n = RB * r + lax.broadcasted_iota(jnp.int32, (RB, 1), 0)
        j = lax.broadcasted_iota(jnp.int32, (1, 128), 1)
        ohh = (jnp.right_shift(n, 7) == j).astype(jnp.float32)
        ohl = ((n & 127) == j).astype(jnp.float32)
        t = jnp.dot(ohh, cnt[...], preferred_element_type=jnp.float32)
        inv = jnp.sum(t * ohl, axis=1, keepdims=True)
        a_lo = slo_ref[0] * inv
        a_hi = shi_ref[0] * inv
        acc = lax.dot_general(a_lo, wl_ref[:, 0:DH],
                              (((1,), (1,)), ((), ())),
                              preferred_element_type=jnp.float32)
        acc += lax.dot_general(a_hi, wl_ref[:, DH:D],
                               (((1,), (1,)), ((), ())),
                               preferred_element_type=jnp.float32)
        acc += lax.dot_general(x_ref[...], wr_ref[...],
                               (((1,), (1,)), ((), ())),
                               preferred_element_type=jnp.float32)
        out_ref[...] = acc + b_ref[...]


def _tc_call(dst2, summed2, x, wl, wr, bias):
    return pl.pallas_call(
        _tc_body,
        grid=(NEB + NMM,),
        in_specs=[
            pl.BlockSpec((EB, 1), lambda i: (jnp.minimum(i, NEB - 1), 0)),
            pl.BlockSpec((1, RB, DH),
                         lambda i: (0, jnp.maximum(i - NEB, 0), 0)),
            pl.BlockSpec((1, RB, DH),
                         lambda i: (1, jnp.maximum(i - NEB, 0), 0)),
            pl.BlockSpec((RB, D), lambda i: (jnp.maximum(i - NEB, 0), 0)),
            pl.BlockSpec((D, D), lambda i: (0, 0)),
            pl.BlockSpec((D, D), lambda i: (0, 0)),
            pl.BlockSpec((1, D), lambda i: (0, 0)),
        ],
        out_specs=pl.BlockSpec((RB, D),
                               lambda i: (jnp.maximum(i - NEB, 0), 0)),
        out_shape=jax.ShapeDtypeStruct((N_NODES, D), jnp.float32),
        scratch_shapes=[pltpu.VMEM((128, 128), jnp.float32)],
    )(dst2, summed2, summed2, x, wl, wr, bias)


@jax.jit
def kernel(x, edge_index, edge_weight, lin_l_w, lin_l_b, lin_r_w, lin_r_b):
    src = edge_index[0].astype(jnp.int32)
    dst = edge_index[1].astype(jnp.int32)
    x_lo = x[:, :DH]
    x_hi = x[:, DH:]

    summed2 = _sc_call(x_lo, x_hi, edge_weight, src, dst)

    bias = (lin_l_b + lin_r_b).reshape(1, D)
    return _tc_call(dst.reshape(N_EDGES, 1), summed2, x, lin_l_w, lin_r_w,
                    bias)


# speedup vs baseline: 1.1808x; 1.1808x over previous
import jax
import jax.numpy as jnp
from jax import lax
from jax.experimental import pallas as pl
from jax.experimental.pallas import tpu as pltpu
from jax.experimental.pallas import tpu_sc as plsc

N_NODES = 10000
N_EDGES = 160000
D = 256
DH = D // 2
NT = 16
EPT = N_EDGES // NT
B = 40
NCH = EPT // B
NBUF = 4
NWT = 10
RPT = N_NODES // NWT
ZR = 40


def _zero_fill(ref, nrows, ncols):
    def row(i, _):
        def col(j, _):
            ref[i, pl.ds(j * 16, 16)] = jnp.zeros((16,), jnp.float32)
            return 0
        lax.fori_loop(0, ncols // 16, col, 0)
        return 0
    lax.fori_loop(0, nrows, row, 0)


def _sc_body(x_lo, x_hi, ew_hbm, src_hbm, dst_hbm,
             summed_out,
             idx_s, idx_d, xg, ewv, zbuf,
             sidx, sdst, sew, sg, ssc,
             acc):
    c = lax.axis_index("c")
    s = lax.axis_index("s")
    row0 = pl.multiple_of(s * RPT, 8)

    @pl.when(s < NWT)
    def _():
        _zero_fill(zbuf, ZR, DH)
        for r in range(0, RPT, ZR):
            pltpu.sync_copy(zbuf, acc.at[pl.ds(row0 + r, ZR)])

    plsc.subcore_barrier()

    ebase = s * EPT

    def process(xref, col0):
        def fetch_src(j, k):
            base = pl.multiple_of(ebase + j * B, 8)
            pltpu.async_copy(src_hbm.at[pl.ds(base, B)], idx_s.at[k],
                             sidx.at[k])

        def fetch_dst(j, k):
            base = pl.multiple_of(ebase + j * B, 8)
            pltpu.async_copy(dst_hbm.at[pl.ds(base, B)], idx_d.at[k],
                             sdst.at[k])

        def fetch_ew(j, k):
            base = pl.multiple_of(ebase + j * B, 8)
            pltpu.async_copy(ew_hbm.at[pl.ds(base, B), pl.ds(col0, DH)],
                             ewv.at[k], sew.at[k])

        def gather_start(k):
            pltpu.make_async_copy(src_hbm.at[pl.ds(0, B)], idx_s.at[k],
                                  sidx.at[k]).wait()
            pltpu.async_copy(xref.at[idx_s.at[k]], xg.at[k], sg.at[k])

        def compute_scatter(k):
            pltpu.make_async_copy(ew_hbm.at[pl.ds(0, B), pl.ds(col0, DH)],
                                  ewv.at[k], sew.at[k]).wait()
            pltpu.make_async_copy(xref.at[pl.ds(0, B)], xg.at[k],
                                  sg.at[k]).wait()

            @plsc.parallel_loop(0, B, 1, unroll=4)
            def _(i):
                for jj in range(DH // 16):
                    sl = pl.ds(jj * 16, 16)
                    ewv[k, i, sl] = ewv[k, i, sl] * xg[k, i, sl]
            pltpu.make_async_copy(dst_hbm.at[pl.ds(0, B)], idx_d.at[k],
                                  sdst.at[k]).wait()
            pltpu.async_copy(ewv.at[k], acc.at[idx_d.at[k]], ssc.at[k],
                             add=True)

        def scatter_wait(k):
            pltpu.make_async_copy(ewv.at[k], acc.at[pl.ds(0, B)],
                                  ssc.at[k]).wait()

        for u in range(NBUF):
            fetch_src(u, u)
        for u in range(NBUF - 1):
            fetch_dst(u, u)
            fetch_ew(u, u)
        for u in range(NBUF - 1):
            gather_start(u)

        def quad(q, _):
            for u in range(NBUF):
                j = NBUF * q + u
                compute_scatter(u)

                @pl.when(j + 3 < NCH)
                def _():
                    gather_start((u + 3) % NBUF)

                @pl.when(j > 0)
                def _():
                    scatter_wait((u + 3) % NBUF)

                @pl.when(j + 3 < NCH)
                def _():
                    fetch_dst(j + 3, (u + 3) % NBUF)
                    fetch_ew(j + 3, (u + 3) % NBUF)

                @pl.when(j + 4 < NCH)
                def _():
                    fetch_src(j + 4, u)
            return 0
        lax.fori_loop(0, NCH // NBUF, quad, 0)
        compute_scatter((NCH - 2) % NBUF)
        scatter_wait((NCH - 3) % NBUF)
        compute_scatter((NCH - 1) % NBUF)
        scatter_wait((NCH - 2) % NBUF)
        scatter_wait((NCH - 1) % NBUF)

    @pl.when(c == 0)
    def _():
        process(x_lo, 0)

    @pl.when(c == 1)
    def _():
        process(x_hi, DH)

    plsc.subcore_barrier()

    @pl.when(s < NWT)
    def _():
        pltpu.sync_copy(acc.at[pl.ds(row0, RPT)],
                        summed_out.at[c, pl.ds(row0, RPT)])


_sc_call = pl.kernel(
    _sc_body,
    out_type=jax.ShapeDtypeStruct((2, N_NODES, DH), jnp.float32),
    mesh=plsc.VectorSubcoreMesh(core_axis_name="c", subcore_axis_name="s"),
    scratch_types=[
        pltpu.VMEM((NBUF, B), jnp.int32),
        pltpu.VMEM((NBUF, B), jnp.int32),
        pltpu.VMEM((NBUF, B, DH), jnp.float32),
        pltpu.VMEM((NBUF, B, DH), jnp.float32),
        pltpu.VMEM((ZR, DH), jnp.float32),
        pltpu.SemaphoreType.DMA((NBUF,)),
        pltpu.SemaphoreType.DMA((NBUF,)),
        pltpu.SemaphoreType.DMA((NBUF,)),
        pltpu.SemaphoreType.DMA((NBUF,)),
        pltpu.SemaphoreType.DMA((NBUF,)),
        pltpu.VMEM_SHARED((N_NODES, DH), jnp.float32),
    ],
    name="gcn_edge_sc",
)


EB = 8000
NEB = N_EDGES // EB
HB = 100


def _cnt_body(dst_ref, out_ref):
    i = pl.program_id(0)

    @pl.when(i == 0)
    def _():
        out_ref[...] = jnp.zeros_like(out_ref)

    val = dst_ref[...]
    j = lax.broadcasted_iota(jnp.int32, (1, 128), 1)
    oh_hi = (val // HB == j).astype(jnp.bfloat16)
    oh_lo = (val % HB == j).astype(jnp.bfloat16)
    out_ref[...] += lax.dot_general(
        oh_hi, oh_lo, (((0,), (0,)), ((), ())),
        preferred_element_type=jnp.float32)

    @pl.when(i == NEB - 1)
    def _():
        out_ref[...] = 1.0 / jnp.clip(out_ref[...], 1.0, None)


def _cnt_call(dst2):
    return pl.pallas_call(
        _cnt_body,
        grid=(NEB,),
        in_specs=[pl.BlockSpec((EB, 1), lambda i: (i, 0))],
        out_specs=pl.BlockSpec((128, 128), lambda i: (0, 0)),
        out_shape=jax.ShapeDtypeStruct((128, 128), jnp.float32),
    )(dst2)


RB = 1000


def _mm_body(slo_ref, shi_ref, inv_ref, x_ref, wl_ref, wr_ref, b_ref,
             out_ref):
    inv = inv_ref[...]
    a_lo = slo_ref[0] * inv
    a_hi = shi_ref[0] * inv
    acc = jnp.dot(a_lo, wl_ref[0:DH, :], preferred_element_type=jnp.float32)
    acc += jnp.dot(a_hi, wl_ref[DH:D, :], preferred_element_type=jnp.float32)
    acc += jnp.dot(x_ref[...], wr_ref[...], preferred_element_type=jnp.float32)
    out_ref[...] = acc + b_ref[...]


def _mm_call(summed2, inv, x, wl_t, wr_t, bias):
    return pl.pallas_call(
        _mm_body,
        grid=(N_NODES // RB,),
        in_specs=[
            pl.BlockSpec((1, RB, DH), lambda i: (0, i, 0)),
            pl.BlockSpec((1, RB, DH), lambda i: (1, i, 0)),
            pl.BlockSpec((RB, 1), lambda i: (i, 0)),
            pl.BlockSpec((RB, D), lambda i: (i, 0)),
            pl.BlockSpec((D, D), lambda i: (0, 0)),
            pl.BlockSpec((D, D), lambda i: (0, 0)),
            pl.BlockSpec((1, D), lambda i: (0, 0)),
        ],
        out_specs=pl.BlockSpec((RB, D), lambda i: (i, 0)),
        out_shape=jax.ShapeDtypeStruct((N_NODES, D), jnp.float32),
    )(summed2, summed2, inv, x, wl_t, wr_t, bias)


@jax.jit
def kernel(x, edge_index, edge_weight, lin_l_w, lin_l_b, lin_r_w, lin_r_b):
    src = edge_index[0].astype(jnp.int32)
    dst = edge_index[1].astype(jnp.int32)
    x_lo = x[:, :DH]
    x_hi = x[:, DH:]

    summed2 = _sc_call(x_lo, x_hi, edge_weight, src, dst)
    inv_mat = _cnt_call(dst.reshape(N_EDGES, 1))
    inv = inv_mat[:HB, :HB].reshape(N_NODES, 1)

    wl_t = lin_l_w.T
    wr_t = lin_r_w.T
    bias = (lin_l_b + lin_r_b).reshape(1, D)
    return _mm_call(summed2, inv, x, wl_t, wr_t, bias)


# speedup vs baseline: 1.1862x; 1.0046x over previous
import jax
import jax.numpy as jnp
from jax import lax
from jax.experimental import pallas as pl
from jax.experimental.pallas import tpu as pltpu
from jax.experimental.pallas import tpu_sc as plsc

N_NODES = 10000
N_EDGES = 160000
D = 256
DH = D // 2
NT = 16
EPT = N_EDGES // NT
B = 40
NCH = EPT // B
NBUF = 4
NWT = 10
RPT = N_NODES // NWT
ZR = 40


def _zero_fill(ref, nrows, ncols):
    def row(i, _):
        def col(j, _):
            ref[i, pl.ds(j * 16, 16)] = jnp.zeros((16,), jnp.float32)
            return 0
        lax.fori_loop(0, ncols // 16, col, 0)
        return 0
    lax.fori_loop(0, nrows, row, 0)


def _sc_body(x_hbm, ew_hbm, src_hbm, dst_hbm,
             summed_out,
             idx_s, idx_d, xg, ewv, zbuf,
             sidx, sdst, sew, sg, ssc,
             acc):
    c = lax.axis_index("c")
    s = lax.axis_index("s")
    row0 = pl.multiple_of(s * RPT, 8)

    @pl.when(s < NWT)
    def _():
        _zero_fill(zbuf, ZR, DH)
        for r in range(0, RPT, ZR):
            pltpu.sync_copy(zbuf, acc.at[pl.ds(row0 + r, ZR)])

    plsc.subcore_barrier()

    ebase = s * EPT

    def process(col0):
        def fetch_src(j, k):
            base = pl.multiple_of(ebase + j * B, 8)
            pltpu.async_copy(src_hbm.at[pl.ds(base, B)], idx_s.at[k],
                             sidx.at[k])

        def fetch_dst(j, k):
            base = pl.multiple_of(ebase + j * B, 8)
            pltpu.async_copy(dst_hbm.at[pl.ds(base, B)], idx_d.at[k],
                             sdst.at[k])

        def fetch_ew(j, k):
            base = pl.multiple_of(ebase + j * B, 8)
            pltpu.async_copy(ew_hbm.at[pl.ds(base, B), pl.ds(col0, DH)],
                             ewv.at[k], sew.at[k])

        def gather_start(k):
            pltpu.make_async_copy(src_hbm.at[pl.ds(0, B)], idx_s.at[k],
                                  sidx.at[k]).wait()
            pltpu.async_copy(
                x_hbm.at[idx_s.at[k], pl.ds(col0, DH)], xg.at[k], sg.at[k])

        def compute_scatter(k):
            pltpu.make_async_copy(ew_hbm.at[pl.ds(0, B), pl.ds(col0, DH)],
                                  ewv.at[k], sew.at[k]).wait()
            pltpu.make_async_copy(x_hbm.at[pl.ds(0, B), pl.ds(col0, DH)],
                                  xg.at[k], sg.at[k]).wait()

            @plsc.parallel_loop(0, B, 1, unroll=4)
            def _(i):
                for jj in range(DH // 16):
                    sl = pl.ds(jj * 16, 16)
                    ewv[k, i, sl] = ewv[k, i, sl] * xg[k, i, sl]
            pltpu.make_async_copy(dst_hbm.at[pl.ds(0, B)], idx_d.at[k],
                                  sdst.at[k]).wait()
            pltpu.async_copy(ewv.at[k], acc.at[idx_d.at[k]], ssc.at[k],
                             add=True)

        def scatter_wait(k):
            pltpu.make_async_copy(ewv.at[k], acc.at[pl.ds(0, B)],
                                  ssc.at[k]).wait()

        for u in range(NBUF):
            fetch_src(u, u)
        for u in range(NBUF - 1):
            fetch_dst(u, u)
            fetch_ew(u, u)
        for u in range(NBUF - 1):
            gather_start(u)

        def quad(q, _):
            for u in range(NBUF):
                j = NBUF * q + u
                compute_scatter(u)

                @pl.when(j + 3 < NCH)
                def _():
                    gather_start((u + 3) % NBUF)

                @pl.when(j > 0)
                def _():
                    scatter_wait((u + 3) % NBUF)

                @pl.when(j + 3 < NCH)
                def _():
                    fetch_dst(j + 3, (u + 3) % NBUF)
                    fetch_ew(j + 3, (u + 3) % NBUF)

                @pl.when(j + 4 < NCH)
                def _():
                    fetch_src(j + 4, u)
            return 0
        lax.fori_loop(0, NCH // NBUF, quad, 0)
        compute_scatter((NCH - 2) % NBUF)
        scatter_wait((NCH - 3) % NBUF)
        compute_scatter((NCH - 1) % NBUF)
        scatter_wait((NCH - 2) % NBUF)
        scatter_wait((NCH - 1) % NBUF)

    @pl.when(c == 0)
    def _():
        process(0)

    @pl.when(c == 1)
    def _():
        process(DH)

    plsc.subcore_barrier()

    @pl.when(s < NWT)
    def _():
        pltpu.sync_copy(acc.at[pl.ds(row0, RPT)],
                        summed_out.at[c, pl.ds(row0, RPT)])


_sc_call = pl.kernel(
    _sc_body,
    out_type=jax.ShapeDtypeStruct((2, N_NODES, DH), jnp.float32),
    mesh=plsc.VectorSubcoreMesh(core_axis_name="c", subcore_axis_name="s"),
    scratch_types=[
        pltpu.VMEM((NBUF, B), jnp.int32),
        pltpu.VMEM((NBUF, B), jnp.int32),
        pltpu.VMEM((NBUF, B, DH), jnp.float32),
        pltpu.VMEM((NBUF, B, DH), jnp.float32),
        pltpu.VMEM((ZR, DH), jnp.float32),
        pltpu.SemaphoreType.DMA((NBUF,)),
        pltpu.SemaphoreType.DMA((NBUF,)),
        pltpu.SemaphoreType.DMA((NBUF,)),
        pltpu.SemaphoreType.DMA((NBUF,)),
        pltpu.SemaphoreType.DMA((NBUF,)),
        pltpu.VMEM_SHARED((N_NODES, DH), jnp.float32),
    ],
    name="gcn_edge_sc",
)


EB = 8000
NEB = N_EDGES // EB
HB = 100


def _cnt_body(dst_ref, out_ref):
    i = pl.program_id(0)

    @pl.when(i == 0)
    def _():
        out_ref[...] = jnp.zeros_like(out_ref)

    val = dst_ref[...]
    j = lax.broadcasted_iota(jnp.int32, (1, 128), 1)
    oh_hi = (val // HB == j).astype(jnp.bfloat16)
    oh_lo = (val % HB == j).astype(jnp.bfloat16)
    out_ref[...] += lax.dot_general(
        oh_hi, oh_lo, (((0,), (0,)), ((), ())),
        preferred_element_type=jnp.float32)

    @pl.when(i == NEB - 1)
    def _():
        out_ref[...] = 1.0 / jnp.clip(out_ref[...], 1.0, None)


def _cnt_call(dst2):
    return pl.pallas_call(
        _cnt_body,
        grid=(NEB,),
        in_specs=[pl.BlockSpec((EB, 1), lambda i: (i, 0))],
        out_specs=pl.BlockSpec((128, 128), lambda i: (0, 0)),
        out_shape=jax.ShapeDtypeStruct((128, 128), jnp.float32),
    )(dst2)


RB = 1000


def _mm_body(slo_ref, shi_ref, inv_ref, x_ref, wl_ref, wr_ref, b_ref,
             out_ref):
    inv = inv_ref[...]
    a_lo = slo_ref[0] * inv
    a_hi = shi_ref[0] * inv
    acc = jnp.dot(a_lo, wl_ref[0:DH, :], preferred_element_type=jnp.float32)
    acc += jnp.dot(a_hi, wl_ref[DH:D, :], preferred_element_type=jnp.float32)
    acc += jnp.dot(x_ref[...], wr_ref[...], preferred_element_type=jnp.float32)
    out_ref[...] = acc + b_ref[...]


def _mm_call(summed2, inv, x, wl_t, wr_t, bias):
    return pl.pallas_call(
        _mm_body,
        grid=(N_NODES // RB,),
        in_specs=[
            pl.BlockSpec((1, RB, DH), lambda i: (0, i, 0)),
            pl.BlockSpec((1, RB, DH), lambda i: (1, i, 0)),
            pl.BlockSpec((RB, 1), lambda i: (i, 0)),
            pl.BlockSpec((RB, D), lambda i: (i, 0)),
            pl.BlockSpec((D, D), lambda i: (0, 0)),
            pl.BlockSpec((D, D), lambda i: (0, 0)),
            pl.BlockSpec((1, D), lambda i: (0, 0)),
        ],
        out_specs=pl.BlockSpec((RB, D), lambda i: (i, 0)),
        out_shape=jax.ShapeDtypeStruct((N_NODES, D), jnp.float32),
    )(summed2, summed2, inv, x, wl_t, wr_t, bias)


@jax.jit
def kernel(x, edge_index, edge_weight, lin_l_w, lin_l_b, lin_r_w, lin_r_b):
    src = edge_index[0].astype(jnp.int32)
    dst = edge_index[1].astype(jnp.int32)
    summed2 = _sc_call(x, edge_weight, src, dst)
    inv_mat = _cnt_call(dst.reshape(N_EDGES, 1))
    inv = inv_mat[:HB, :HB].reshape(N_NODES, 1)

    wl_t = lin_l_w.T
    wr_t = lin_r_w.T
    bias = (lin_l_b + lin_r_b).reshape(1, D)
    return _mm_call(summed2, inv, x, wl_t, wr_t, bias)
